# baseline (speedup 1.0000x reference)
import functools
import jax
import jax.numpy as jnp
from jax import lax
from jax.experimental import pallas as pl
from jax.experimental.pallas import tpu as pltpu
from jax.experimental.pallas import tpu_sc as plsc

_K = 19
_B = 8
_NPIX = 512 * 512
_NC = 2
_NS = 16
_NW = _NC * _NS
_P = 1024
_QP = _NPIX // 4
_ROUNDS = _QP // _P
_ACC = 16 * 1024


def _sc_body(l_hbm, t_hbm, out_hbm, bufA, tbA, bufB, tbB, acc,
             semA, semB):
    cidx = lax.axis_index("c")
    sidx = lax.axis_index("s")
    w = cidx * _NS + sidx
    b = w // 4
    q = w % 4
    lbase = b * (_K * _NPIX)
    tbase = b * _NPIX
    qbase = q * _QP

    zeros16 = jnp.zeros((16,), jnp.float32)

    def zloop(i, carry):
        acc[pl.ds(i * 16, 16)] = zeros16
        return carry

    lax.fori_loop(0, 4 * _ACC // 16, zloop, 0)

    def fire(r, buf, tb, sem):
        p0 = qbase + r * _P
        for c in range(_K):
            pltpu.async_copy(l_hbm.at[pl.ds(lbase + c * _NPIX + p0, _P)],
                             buf.at[pl.ds(c * _P, _P)], sem)
        pltpu.async_copy(t_hbm.at[pl.ds(tbase + p0, _P)], tb, sem)

    def drain(r, buf, tb, sem):
        p0 = qbase + r * _P
        for c in range(_K):
            pltpu.make_async_copy(
                l_hbm.at[pl.ds(lbase + c * _NPIX + p0, _P)],
                buf.at[pl.ds(c * _P, _P)], sem).wait()
        pltpu.make_async_copy(t_hbm.at[pl.ds(tbase + p0, _P)], tb,
                              sem).wait()

    lanes = lax.iota(jnp.int32, 16) * 1024
    ones16 = jnp.ones((16,), jnp.float32)
    subtab = [jnp.full((16,), u * _ACC, jnp.int32) for u in range(4)]

    def compute(buf, tb):
        def inner(j, carry):
            for u in range(4):
                o = (j * 4 + u) * 16
                best = buf[pl.ds(o, 16)]
                pred = jnp.zeros((16,), jnp.int32)
                for c in range(1, _K):
                    v = buf[pl.ds(c * _P + o, 16)]
                    m = v > best
                    best = jnp.where(m, v, best)
                    pred = jnp.where(m, jnp.full((16,), c, jnp.int32), pred)
                t = tb[pl.ds(o, 16)]
                addr = subtab[u] + lanes + t * 32 + pred
                plsc.addupdate_scatter(acc, [addr], ones16)
                addr2 = subtab[u] + lanes + (19 * 32) + t
                plsc.addupdate_scatter(acc, [addr2], ones16)
            return carry

        lax.fori_loop(0, _P // 64, inner, 0)

    fire(0, bufA, tbA, semA)
    fire(1, bufB, tbB, semB)

    def round2(i, carry):
        g = i * 2
        drain(g, bufA, tbA, semA)
        compute(bufA, tbA)

        @pl.when(g + 2 < _ROUNDS)
        def _():
            fire(g + 2, bufA, tbA, semA)

        drain(g + 1, bufB, tbB, semB)
        compute(bufB, tbB)

        @pl.when(g + 3 < _ROUNDS)
        def _():
            fire(g + 3, bufB, tbB, semB)

        return carry

    lax.fori_loop(0, _ROUNDS // 2, round2, 0)

    def fold(i, carry):
        o = i * 16
        s = (acc[pl.ds(o, 16)] + acc[pl.ds(_ACC + o, 16)]
             + acc[pl.ds(2 * _ACC + o, 16)] + acc[pl.ds(3 * _ACC + o, 16)])
        acc[pl.ds(o, 16)] = s
        return carry

    lax.fori_loop(0, _ACC // 16, fold, 0)
    pltpu.sync_copy(acc.at[pl.ds(0, _ACC)], out_hbm.at[pl.ds(w * _ACC, _ACC)])


def _sc_hist(lflat, tflat):
    mesh = plsc.VectorSubcoreMesh(core_axis_name="c", subcore_axis_name="s")
    k = functools.partial(
        pl.kernel,
        mesh=mesh,
        out_type=jax.ShapeDtypeStruct((_NW * _ACC,), jnp.float32),
        compiler_params=pltpu.CompilerParams(use_tc_tiling_on_sc=False,
                                             needs_layout_passes=False),
        scratch_types=[
            pltpu.VMEM((_K * _P,), jnp.float32),
            pltpu.VMEM((_P,), jnp.int32),
            pltpu.VMEM((_K * _P,), jnp.float32),
            pltpu.VMEM((_P,), jnp.int32),
            pltpu.VMEM((4 * _ACC,), jnp.float32),
            pltpu.SemaphoreType.DMA,
            pltpu.SemaphoreType.DMA,
        ],
    )(_sc_body)
    return k(lflat, tflat)


def _fin_body(p_ref, inter_ref, union_ref, total_ref, freq_ref):
    x = p_ref[...]
    cm = jnp.sum(x, axis=0)
    cm19 = cm[0:_K]
    r0 = lax.broadcasted_iota(jnp.int32, (_K, 32), 0)
    r1 = lax.broadcasted_iota(jnp.int32, (_K, 32), 1)
    eye = (r0 == r1).astype(jnp.float32)
    inter = jnp.sum(cm19 * eye, axis=0, keepdims=True)
    cols = jnp.sum(cm19, axis=0, keepdims=True)
    rows = cm[_K:_K + 1, :]
    total = jnp.sum(rows)
    inter_ref[...] = inter[:, 0:_K]
    union_ref[...] = (rows + cols - inter)[:, 0:_K]
    total_ref[...] = jnp.reshape(total, (1, 1))
    freq_ref[...] = (rows / total)[:, 0:_K]


def _finalize(parts):
    vec = jax.ShapeDtypeStruct((1, _K), jnp.float32)
    return pl.pallas_call(
        _fin_body,
        out_shape=[vec, vec, jax.ShapeDtypeStruct((1, 1), jnp.float32), vec],
    )(parts)


def kernel(logits, target):
    parts = _sc_hist(logits.reshape(-1), target.reshape(-1))
    inter, union, total, freq = _finalize(parts.reshape(_NW * 16, 32, 32))
    return (inter.reshape(_K), union.reshape(_K),
            total.reshape(()), freq.reshape(_K))


# speedup vs baseline: 1.0906x; 1.0906x over previous
import functools
import jax
import jax.numpy as jnp
from jax import lax
from jax.experimental import pallas as pl
from jax.experimental.pallas import tpu as pltpu
from jax.experimental.pallas import tpu_sc as plsc

_K = 19
_B = 8
_NPIX = 512 * 512
_NC = 2
_NS = 16
_NW = _NC * _NS
_P = 2048
_QP = _NPIX // 4
_ROUNDS = _QP // _P
_ACC = 16 * 1024


def _sc_body(l_hbm, t_hbm, out_hbm, bufA, tbA, bufB, tbB, acc,
             semA, semB):
    cidx = lax.axis_index("c")
    sidx = lax.axis_index("s")
    w = cidx * _NS + sidx
    b = w // 4
    q = w % 4
    rbase = b * _K
    tbase = b * _NPIX
    qbase = q * _QP

    zeros16 = jnp.zeros((16,), jnp.float32)

    def zloop(i, carry):
        acc[pl.ds(i * 16, 16)] = zeros16
        return carry

    lax.fori_loop(0, _ACC // 16, zloop, 0)

    def fire(r, buf, tb, sem):
        p0 = qbase + r * _P
        pltpu.async_copy(l_hbm.at[pl.ds(rbase, _K), pl.ds(p0, _P)], buf, sem)
        pltpu.async_copy(t_hbm.at[pl.ds(tbase + p0, _P)], tb, sem)

    def drain(r, buf, tb, sem):
        p0 = qbase + r * _P
        pltpu.make_async_copy(
            l_hbm.at[pl.ds(rbase, _K), pl.ds(p0, _P)], buf, sem).wait()
        pltpu.make_async_copy(t_hbm.at[pl.ds(tbase + p0, _P)], tb,
                              sem).wait()

    lanes = lax.iota(jnp.int32, 16) * 1024
    lanes_r = lanes + _K * 32
    ones16 = jnp.ones((16,), jnp.float32)

    def compute(buf, tb):
        def inner(j, carry):
            for u in range(4):
                o = (j * 4 + u) * 16
                best = buf[0, pl.ds(o, 16)]
                pred = jnp.zeros((16,), jnp.int32)
                for c in range(1, _K):
                    v = buf[c, pl.ds(o, 16)]
                    m = v > best
                    best = jnp.where(m, v, best)
                    pred = jnp.where(m, jnp.full((16,), c, jnp.int32), pred)
                t = tb[pl.ds(o, 16)]
                addr = lanes + t * 32 + pred
                plsc.addupdate_scatter(acc, [addr], ones16)
                addr2 = lanes_r + t
                plsc.addupdate_scatter(acc, [addr2], ones16)
            return carry

        lax.fori_loop(0, _P // 64, inner, 0)

    fire(0, bufA, tbA, semA)
    fire(1, bufB, tbB, semB)

    def round2(i, carry):
        g = i * 2
        drain(g, bufA, tbA, semA)
        compute(bufA, tbA)

        @pl.when(g + 2 < _ROUNDS)
        def _():
            fire(g + 2, bufA, tbA, semA)

        drain(g + 1, bufB, tbB, semB)
        compute(bufB, tbB)

        @pl.when(g + 3 < _ROUNDS)
        def _():
            fire(g + 3, bufB, tbB, semB)

        return carry

    lax.fori_loop(0, _ROUNDS // 2, round2, 0)

    pltpu.sync_copy(acc, out_hbm.at[pl.ds(w * _ACC, _ACC)])


def _sc_hist(l2d, tflat):
    mesh = plsc.VectorSubcoreMesh(core_axis_name="c", subcore_axis_name="s")
    k = functools.partial(
        pl.kernel,
        mesh=mesh,
        out_type=jax.ShapeDtypeStruct((_NW * _ACC,), jnp.float32),
        compiler_params=pltpu.CompilerParams(use_tc_tiling_on_sc=False,
                                             needs_layout_passes=False),
        scratch_types=[
            pltpu.VMEM((_K, _P), jnp.float32),
            pltpu.VMEM((_P,), jnp.int32),
            pltpu.VMEM((_K, _P), jnp.float32),
            pltpu.VMEM((_P,), jnp.int32),
            pltpu.VMEM((_ACC,), jnp.float32),
            pltpu.SemaphoreType.DMA,
            pltpu.SemaphoreType.DMA,
        ],
    )(_sc_body)
    return k(l2d, tflat)


def _fin_body(p_ref, inter_ref, union_ref, total_ref, freq_ref):
    x = p_ref[...]
    cm = jnp.sum(x, axis=0)
    cm19 = cm[0:_K]
    r0 = lax.broadcasted_iota(jnp.int32, (_K, 32), 0)
    r1 = lax.broadcasted_iota(jnp.int32, (_K, 32), 1)
    eye = (r0 == r1).astype(jnp.float32)
    inter = jnp.sum(cm19 * eye, axis=0, keepdims=True)
    cols = jnp.sum(cm19, axis=0, keepdims=True)
    rows = cm[_K:_K + 1, :]
    total = jnp.sum(rows)
    inter_ref[...] = inter[:, 0:_K]
    union_ref[...] = (rows + cols - inter)[:, 0:_K]
    total_ref[...] = jnp.reshape(total, (1, 1))
    freq_ref[...] = (rows / total)[:, 0:_K]


def _finalize(parts):
    vec = jax.ShapeDtypeStruct((1, _K), jnp.float32)
    return pl.pallas_call(
        _fin_body,
        out_shape=[vec, vec, jax.ShapeDtypeStruct((1, 1), jnp.float32), vec],
    )(parts)


def kernel(logits, target):
    parts = _sc_hist(logits.reshape(_B * _K, _NPIX), target.reshape(-1))
    inter, union, total, freq = _finalize(parts.reshape(_NW * 16, 32, 32))
    return (inter.reshape(_K), union.reshape(_K),
            total.reshape(()), freq.reshape(_K))
